# baseline (speedup 1.0000x reference)
import functools

import jax
import jax.numpy as jnp
from jax import lax
from jax.experimental import pallas as pl
from jax.experimental.pallas import tpu as pltpu
from jax.experimental.pallas import tpu_sc as plsc

NUM_VIRTUAL_TOKENS = 200
TOKEN_DIM = 128
BATCH = 1024

NC = 2
NS = 16
NW = NC * NS

B_TOTAL = BATCH * NUM_VIRTUAL_TOKENS
B_PER_W = B_TOTAL // NW
CHUNK = 400
N_CHUNKS = B_PER_W // CHUNK
NBUF = 2


@functools.partial(
    pl.kernel,
    out_type=jax.ShapeDtypeStruct((B_TOTAL, TOKEN_DIM), jnp.float32),
    mesh=plsc.VectorSubcoreMesh(
        core_axis_name="c", subcore_axis_name="s", num_cores=NC,
        num_subcores=NS),
    scratch_types=[
        [pltpu.VMEM((CHUNK,), jnp.int32) for _ in range(NBUF)],
        [pltpu.VMEM((CHUNK, TOKEN_DIM), jnp.float32) for _ in range(NBUF)],
        pltpu.SemaphoreType.DMA,
        pltpu.SemaphoreType.DMA,
    ],
)
def _gather_kernel(idx_hbm, table_hbm, out_hbm, idx_v, rows_v, g_sem, s_sem):
    wid = lax.axis_index("s") * NC + lax.axis_index("c")
    out_base = wid * B_PER_W

    def start_gather(i):
        b = i % NBUF
        pltpu.sync_copy(idx_hbm.at[wid, i], idx_v[b])
        return pltpu.async_copy(table_hbm.at[idx_v[b]], rows_v[b], g_sem)

    gathers = [None] * N_CHUNKS
    scatters = [None] * N_CHUNKS
    gathers[0] = start_gather(0)
    for i in range(N_CHUNKS):
        b = i % NBUF
        if i >= NBUF - 1 and i + 1 < N_CHUNKS:
            scatters[i + 1 - NBUF].wait()
        if i + 1 < N_CHUNKS:
            gathers[i + 1] = start_gather(i + 1)
        gathers[i].wait()
        scatters[i] = pltpu.async_copy(
            rows_v[b], out_hbm.at[pl.ds(out_base + i * CHUNK, CHUNK)], s_sem)
    for i in range(N_CHUNKS - NBUF + 1, N_CHUNKS):
        scatters[i].wait()


def kernel(indices, W):
    idx = indices.reshape(NW, N_CHUNKS, CHUNK).astype(jnp.int32)
    out = _gather_kernel(idx, W)
    return out.reshape(BATCH, NUM_VIRTUAL_TOKENS, TOKEN_DIM)


# speedup vs baseline: 3.9741x; 3.9741x over previous
import functools

import jax
import jax.numpy as jnp
from jax import lax
from jax.experimental import pallas as pl
from jax.experimental.pallas import tpu as pltpu
from jax.experimental.pallas import tpu_sc as plsc

NUM_VIRTUAL_TOKENS = 200
TOKEN_DIM = 128
BATCH = 1024

NC = 2
NS = 16
NW = NC * NS

B_TOTAL = BATCH * NUM_VIRTUAL_TOKENS
B_PER_W = B_TOTAL // NW
CHUNK = 400
N_CHUNKS = B_PER_W // CHUNK
NBUF = 2


@functools.partial(
    pl.kernel,
    out_type=jax.ShapeDtypeStruct((B_TOTAL, TOKEN_DIM), jnp.float32),
    mesh=plsc.VectorSubcoreMesh(
        core_axis_name="c", subcore_axis_name="s", num_cores=NC,
        num_subcores=NS),
    scratch_types=[
        pltpu.VMEM_SHARED((NUM_VIRTUAL_TOKENS, TOKEN_DIM), jnp.float32),
        [pltpu.VMEM((CHUNK,), jnp.int32) for _ in range(NBUF)],
        [pltpu.VMEM((CHUNK, TOKEN_DIM), jnp.float32) for _ in range(NBUF)],
        pltpu.SemaphoreType.DMA,
        pltpu.SemaphoreType.DMA,
    ],
)
def _gather_kernel(idx_hbm, table_hbm, out_hbm, w_v, idx_v, rows_v, g_sem,
                   s_sem):
    wid = lax.axis_index("s") * NC + lax.axis_index("c")
    out_base = wid * B_PER_W

    @pl.when(lax.axis_index("s") == 0)
    def _():
        pltpu.sync_copy(table_hbm, w_v)

    plsc.subcore_barrier()

    def start_gather(i):
        b = i % NBUF
        pltpu.sync_copy(idx_hbm.at[wid, i], idx_v[b])
        return pltpu.async_copy(w_v.at[idx_v[b]], rows_v[b], g_sem)

    gathers = [None] * N_CHUNKS
    scatters = [None] * N_CHUNKS
    gathers[0] = start_gather(0)
    for i in range(N_CHUNKS):
        b = i % NBUF
        if i >= NBUF - 1 and i + 1 < N_CHUNKS:
            scatters[i + 1 - NBUF].wait()
        if i + 1 < N_CHUNKS:
            gathers[i + 1] = start_gather(i + 1)
        gathers[i].wait()
        scatters[i] = pltpu.async_copy(
            rows_v[b], out_hbm.at[pl.ds(out_base + i * CHUNK, CHUNK)], s_sem)
    for i in range(N_CHUNKS - NBUF + 1, N_CHUNKS):
        scatters[i].wait()


def kernel(indices, W):
    idx = indices.reshape(NW, N_CHUNKS, CHUNK).astype(jnp.int32)
    out = _gather_kernel(idx, W)
    return out.reshape(BATCH, NUM_VIRTUAL_TOKENS, TOKEN_DIM)
